# baseline (speedup 1.0000x reference)
import functools

import jax
import jax.numpy as jnp
from jax import lax
from jax.experimental import pallas as pl
from jax.experimental.pallas import tpu as pltpu
from jax.experimental.pallas import tpu_sc as plsc

_NC = 2
_NS = 16
_NW = _NC * _NS

_BATCH = 4096
_BAG = 50
_DIM = 64
_BAGS_PER_W = _BATCH // _NW
_BAGS_PER_CHUNK = 2
_CHUNK = _BAGS_PER_CHUNK * _BAG
_NCHUNKS = _BAGS_PER_W // _BAGS_PER_CHUNK


def _bag_reduce(rows_ref, feat_ref, first_bag):
    for b in range(_BAGS_PER_CHUNK):
        base = b * _BAG
        for cc in range(_DIM // 16):
            col = pl.ds(cc * 16, 16)
            acc = rows_ref[base, col]
            for r in range(1, _BAG):
                acc = acc + rows_ref[base + r, col]
            feat_ref[first_bag + b, col] = acc


def _embedding_bag_sc(bow3, emb_weight):
    mesh = plsc.VectorSubcoreMesh(core_axis_name="c", subcore_axis_name="s",
                                  num_cores=_NC, num_subcores=_NS)

    @functools.partial(
        pl.kernel,
        out_type=jax.ShapeDtypeStruct((_BATCH, _DIM), jnp.float32),
        mesh=mesh,
        scratch_types=[
            pltpu.VMEM((_NCHUNKS, _CHUNK), jnp.int32),
            pltpu.VMEM((_CHUNK, _DIM), jnp.float32),
            pltpu.VMEM((_CHUNK, _DIM), jnp.float32),
            pltpu.VMEM((_BAGS_PER_W, _DIM), jnp.float32),
            pltpu.SemaphoreType.DMA,
            pltpu.SemaphoreType.DMA,
        ],
        compiler_params=pltpu.CompilerParams(use_tc_tiling_on_sc=False),
    )
    def k(bow_hbm, table_hbm, out_hbm, idx_v, rows_a, rows_b, feat_v,
          sem_a, sem_b):
        wid = lax.axis_index("s") * _NC + lax.axis_index("c")
        pltpu.sync_copy(bow_hbm.at[wid], idx_v)
        pltpu.async_copy(table_hbm.at[idx_v.at[0]], rows_a, sem_a)

        def step(i, carry):
            pltpu.make_async_copy(table_hbm.at[idx_v.at[2 * i]],
                                  rows_a, sem_a).wait()
            pltpu.async_copy(table_hbm.at[idx_v.at[2 * i + 1]], rows_b, sem_b)
            _bag_reduce(rows_a, feat_v, 4 * i)

            @pl.when(i < _NCHUNKS // 2 - 1)
            def _():
                pltpu.async_copy(table_hbm.at[idx_v.at[2 * i + 2]],
                                 rows_a, sem_a)

            pltpu.make_async_copy(table_hbm.at[idx_v.at[2 * i + 1]],
                                  rows_b, sem_b).wait()
            _bag_reduce(rows_b, feat_v, 4 * i + 2)
            return carry

        lax.fori_loop(0, _NCHUNKS // 2, step, 0)
        pltpu.sync_copy(feat_v, out_hbm.at[pl.ds(wid * _BAGS_PER_W,
                                                 _BAGS_PER_W)])

    return k(bow3, emb_weight)


def _classifier_tc(features, W, b2):
    def body(f_ref, w_ref, b_ref, o_ref):
        f = f_ref[...]
        w = w_ref[...]
        logits = lax.dot_general(f, w, (((1,), (1,)), ((), ())),
                                 preferred_element_type=jnp.float32)
        logits = logits + b_ref[...]
        m = jnp.max(logits, axis=1, keepdims=True)
        e = jnp.exp(logits - m)
        s = jnp.sum(e, axis=1, keepdims=True)
        o_ref[...] = logits - m - jnp.log(s)

    return pl.pallas_call(
        body,
        out_shape=jax.ShapeDtypeStruct((_BATCH, W.shape[0]), jnp.float32),
    )(features, W, b2)


@jax.jit
def kernel(bow, emb_weight, W, b):
    bow3 = bow.reshape(_NW, _NCHUNKS, _CHUNK)
    features = _embedding_bag_sc(bow3, emb_weight)
    return _classifier_tc(features, W, b.reshape(1, -1))
